# Initial kernel scaffold; baseline (speedup 1.0000x reference)
import jax
import jax.numpy as jnp
from jax.experimental import pallas as pl


def kernel(X, edge_index, W0, W1):
    raise NotImplementedError("write your pallas kernel here")


# speedup vs baseline: 19.0151x; 19.0151x over previous
import functools

import jax
import jax.numpy as jnp
from jax import lax
from jax.experimental import pallas as pl
from jax.experimental.pallas import tpu as pltpu
from jax.experimental.pallas import tpu_sc as plsc

_N = 10000
_E = 320000
_DF = 128
_DC = 16
_NC = 2
_NS = 16
_NW = _NC * _NS
_CHUNK = 80
_EPT = _E // _NW
_NCHUNK = _EPT // _CHUNK
_RPT = 632
_NPAD = _RPT * _NS

_mesh = plsc.VectorSubcoreMesh(core_axis_name="c", subcore_axis_name="s")


def _hist_body(rows_hbm, ones_hbm, z_hbm, out_hbm, ridx, buf, acc, sem):
    del sem
    cid = lax.axis_index("c")
    sid = lax.axis_index("s")
    wid = cid * _NS + sid
    r0 = sid * _RPT
    pltpu.sync_copy(z_hbm.at[pl.ds(r0, _RPT)], acc.at[pl.ds(r0, _RPT)])
    pltpu.sync_copy(ones_hbm, buf)
    pltpu.sync_copy(rows_hbm.at[wid], ridx)
    plsc.subcore_barrier()

    @pl.loop(0, _NCHUNK)
    def _(i):
        pltpu.sync_copy(buf, acc.at[ridx.at[i]], add=True)

    plsc.subcore_barrier()
    pltpu.sync_copy(acc.at[pl.ds(r0, _RPT)], out_hbm.at[cid, pl.ds(r0, _RPT)])


@functools.partial(
    pl.kernel,
    out_type=jax.ShapeDtypeStruct((_NC, _NPAD, _DC), jnp.float32),
    mesh=_mesh,
    scratch_types=[
        pltpu.VMEM((_NCHUNK, _CHUNK), jnp.int32),
        pltpu.VMEM((_CHUNK, _DC), jnp.float32),
        pltpu.VMEM_SHARED((_NPAD, _DC), jnp.float32),
        pltpu.SemaphoreType.DMA,
    ],
)
def _hist(rows_hbm, ones_hbm, z_hbm, out_hbm, ridx, buf, acc, sem):
    _hist_body(rows_hbm, ones_hbm, z_hbm, out_hbm, ridx, buf, acc, sem)


def _spmm_body(d, rows_hbm, cols_hbm, g_hbm, z_hbm, out_hbm,
               ridx, cidx, buf, acc, sem):
    cid = lax.axis_index("c")
    sid = lax.axis_index("s")
    wid = cid * _NS + sid
    r0 = sid * _RPT
    pltpu.sync_copy(z_hbm.at[pl.ds(r0, _RPT)], acc.at[pl.ds(r0, _RPT)])
    pltpu.sync_copy(rows_hbm.at[wid], ridx)
    pltpu.sync_copy(cols_hbm.at[wid], cidx)
    plsc.subcore_barrier()

    @pl.loop(0, _NCHUNK)
    def _(i):
        pltpu.async_copy(g_hbm.at[cidx.at[i]], buf, sem).wait()
        pltpu.sync_copy(buf, acc.at[ridx.at[i]], add=True)

    plsc.subcore_barrier()
    pltpu.sync_copy(acc.at[pl.ds(r0, _RPT)], out_hbm.at[cid, pl.ds(r0, _RPT)])
    del d


def _make_spmm(d):
    @functools.partial(
        pl.kernel,
        out_type=jax.ShapeDtypeStruct((_NC, _NPAD, d), jnp.float32),
        mesh=_mesh,
        scratch_types=[
            pltpu.VMEM((_NCHUNK, _CHUNK), jnp.int32),
            pltpu.VMEM((_NCHUNK, _CHUNK), jnp.int32),
            pltpu.VMEM((_CHUNK, d), jnp.float32),
            pltpu.VMEM_SHARED((_NPAD, d), jnp.float32),
            pltpu.SemaphoreType.DMA,
        ],
    )
    def spmm(rows_hbm, cols_hbm, g_hbm, z_hbm, out_hbm,
             ridx, cidx, buf, acc, sem):
        _spmm_body(d, rows_hbm, cols_hbm, g_hbm, z_hbm, out_hbm,
                   ridx, cidx, buf, acc, sem)

    return spmm


_spmm128 = _make_spmm(_DF)

_BLK = 2000


def _dis_of(dacc_ref):
    deg = dacc_ref[0, :, 0:1] + dacc_ref[1, :, 0:1] + 1.0
    return lax.rsqrt(deg)


def _mm0_body(x_ref, w0t_ref, dacc_ref, g0_ref):
    dis = _dis_of(dacc_ref)
    h = jnp.dot(x_ref[...], w0t_ref[...], preferred_element_type=jnp.float32)
    g0_ref[...] = h * dis


def _mid_body(p0_ref, g0_ref, dacc_ref, g1_ref):
    dis = _dis_of(dacc_ref)
    s0 = (p0_ref[0] + p0_ref[1] + g0_ref[...]) * dis
    g1_ref[...] = jnp.maximum(s0, 0.0) * dis


def _fin_body(p1_ref, g1_ref, dacc_ref, w1t_ref, out_ref):
    dis = _dis_of(dacc_ref)
    s1 = (p1_ref[0] + p1_ref[1] + g1_ref[...]) * dis
    out_ref[...] = jnp.dot(s1, w1t_ref[...],
                           preferred_element_type=jnp.float32)


def _row_spec(d):
    return pl.BlockSpec((_BLK, d), lambda i: (i, 0))


def _pair_spec(d):
    return pl.BlockSpec((_NC, _BLK, d), lambda i: (0, i, 0))


def _full_spec(r, c):
    return pl.BlockSpec((r, c), lambda i: (0, 0))


def _mm0(X, W0T, dacc):
    return pl.pallas_call(
        _mm0_body,
        grid=(_N // _BLK,),
        in_specs=[_row_spec(_DF), _full_spec(_DF, _DF), _pair_spec(_DC)],
        out_specs=_row_spec(_DF),
        out_shape=jax.ShapeDtypeStruct((_N, _DF), jnp.float32),
    )(X, W0T, dacc)


def _mid(p0, g0, dacc):
    return pl.pallas_call(
        _mid_body,
        grid=(_N // _BLK,),
        in_specs=[_pair_spec(_DF), _row_spec(_DF), _pair_spec(_DC)],
        out_specs=_row_spec(_DF),
        out_shape=jax.ShapeDtypeStruct((_N, _DF), jnp.float32),
    )(p0, g0, dacc)


def _fin(p1, g1, dacc, W1T):
    return pl.pallas_call(
        _fin_body,
        grid=(_N // _BLK,),
        in_specs=[_pair_spec(_DF), _row_spec(_DF), _pair_spec(_DC),
                  _full_spec(_DF, _DC)],
        out_specs=_row_spec(_DC),
        out_shape=jax.ShapeDtypeStruct((_N, _DC), jnp.float32),
    )(p1, g1, dacc, W1T)


def kernel(X, edge_index, W0, W1):
    rows = edge_index[0].reshape(_NW, _NCHUNK, _CHUNK)
    cols = edge_index[1].reshape(_NW, _NCHUNK, _CHUNK)
    ones = jnp.ones((_CHUNK, _DC), jnp.float32)
    z16 = jnp.zeros((_NPAD, _DC), jnp.float32)
    z128 = jnp.zeros((_NPAD, _DF), jnp.float32)

    dacc = _hist(rows, ones, z16)
    g0 = _mm0(X, W0.T, dacc)
    p0 = _spmm128(rows, cols, g0, z128)
    g1 = _mid(p0, g0, dacc)
    p1 = _spmm128(rows, cols, g1, z128)
    return _fin(p1, g1, dacc, W1.T)


# speedup vs baseline: 28.8831x; 1.5190x over previous
import functools

import jax
import jax.numpy as jnp
from jax import lax
from jax.experimental import pallas as pl
from jax.experimental.pallas import tpu as pltpu
from jax.experimental.pallas import tpu_sc as plsc

_N = 10000
_E = 320000
_DF = 128
_DC = 16
_NC = 2
_NS = 16
_NW = _NC * _NS
_CHUNK = 125
_EPT = _E // _NW
_NCHUNK = _EPT // _CHUNK
_CPP = 16
_RPT = 632
_NPAD = _RPT * _NS

_mesh = plsc.VectorSubcoreMesh(core_axis_name="c", subcore_axis_name="s")


def _hist_body(rows_hbm, ones_hbm, z_hbm, out_hbm, ridx, buf, acc, sem):
    del sem
    cid = lax.axis_index("c")
    sid = lax.axis_index("s")
    wid = cid * _NS + sid
    r0 = sid * _RPT
    pltpu.sync_copy(z_hbm.at[pl.ds(r0, _RPT)], acc.at[pl.ds(r0, _RPT)])
    pltpu.sync_copy(ones_hbm, buf)
    pltpu.sync_copy(rows_hbm.at[wid], ridx)
    plsc.subcore_barrier()

    @pl.loop(0, _NCHUNK)
    def _(i):
        pltpu.sync_copy(buf, acc.at[ridx.at[i]], add=True)

    plsc.subcore_barrier()
    pltpu.sync_copy(acc.at[pl.ds(r0, _RPT)], out_hbm.at[cid, pl.ds(r0, _RPT)])


@functools.partial(
    pl.kernel,
    out_type=jax.ShapeDtypeStruct((_NC, _NPAD, _DC), jnp.float32),
    mesh=_mesh,
    scratch_types=[
        pltpu.VMEM((_NCHUNK, _CHUNK), jnp.int32),
        pltpu.VMEM((_CHUNK, _DC), jnp.float32),
        pltpu.VMEM_SHARED((_NPAD, _DC), jnp.float32),
        pltpu.SemaphoreType.DMA,
    ],
)
def _hist(rows_hbm, ones_hbm, z_hbm, out_hbm, ridx, buf, acc, sem):
    _hist_body(rows_hbm, ones_hbm, z_hbm, out_hbm, ridx, buf, acc, sem)


def _spmm_body(d, rows_hbm, cols_hbm, g_hbm, z_hbm, out_hbm,
               ridx, cidx, buf0, buf1, acc, sem0, sem1):
    cid = lax.axis_index("c")
    sid = lax.axis_index("s")
    wid = cid * _NS + sid
    r0 = sid * _RPT
    pltpu.sync_copy(z_hbm.at[pl.ds(r0, _RPT)], acc.at[pl.ds(r0, _RPT)])
    plsc.subcore_barrier()

    nprs = _CPP // 2
    for p in range(_NCHUNK // _CPP):
        pltpu.sync_copy(rows_hbm.at[wid, pl.ds(p * _CPP, _CPP)], ridx)
        pltpu.sync_copy(cols_hbm.at[wid, pl.ds(p * _CPP, _CPP)], cidx)
        pltpu.async_copy(g_hbm.at[cidx.at[0]], buf0, sem0)

        @pl.loop(0, nprs)
        def _(j):
            i = 2 * j
            pltpu.async_copy(g_hbm.at[cidx.at[i + 1]], buf1, sem1)
            pltpu.make_async_copy(g_hbm.at[cidx.at[i]], buf0, sem0).wait()
            pltpu.sync_copy(buf0, acc.at[ridx.at[i]], add=True)

            @pl.when(j < nprs - 1)
            def _():
                pltpu.async_copy(g_hbm.at[cidx.at[i + 2]], buf0, sem0)

            pltpu.make_async_copy(g_hbm.at[cidx.at[i + 1]], buf1, sem1).wait()
            pltpu.sync_copy(buf1, acc.at[ridx.at[i + 1]], add=True)

    plsc.subcore_barrier()
    pltpu.sync_copy(acc.at[pl.ds(r0, _RPT)], out_hbm.at[cid, pl.ds(r0, _RPT)])
    del d


def _make_spmm(d):
    @functools.partial(
        pl.kernel,
        out_type=jax.ShapeDtypeStruct((_NC, _NPAD, d), jnp.float32),
        mesh=_mesh,
        scratch_types=[
            pltpu.VMEM((_CPP, _CHUNK), jnp.int32),
            pltpu.VMEM((_CPP, _CHUNK), jnp.int32),
            pltpu.VMEM((_CHUNK, d), jnp.float32),
            pltpu.VMEM((_CHUNK, d), jnp.float32),
            pltpu.VMEM_SHARED((_NPAD, d), jnp.float32),
            pltpu.SemaphoreType.DMA,
            pltpu.SemaphoreType.DMA,
        ],
    )
    def spmm(rows_hbm, cols_hbm, g_hbm, z_hbm, out_hbm,
             ridx, cidx, buf0, buf1, acc, sem0, sem1):
        _spmm_body(d, rows_hbm, cols_hbm, g_hbm, z_hbm, out_hbm,
                   ridx, cidx, buf0, buf1, acc, sem0, sem1)

    return spmm


_spmm128 = _make_spmm(_DF)

_BLK = 2000


def _dis_of(dacc_ref):
    deg = dacc_ref[0, :, 0:1] + dacc_ref[1, :, 0:1] + 1.0
    return lax.rsqrt(deg)


def _mm0_body(x_ref, w0t_ref, dacc_ref, g0_ref):
    dis = _dis_of(dacc_ref)
    h = jnp.dot(x_ref[...], w0t_ref[...], preferred_element_type=jnp.float32)
    g0_ref[...] = h * dis


def _mid_body(p0_ref, g0_ref, dacc_ref, g1_ref):
    dis = _dis_of(dacc_ref)
    s0 = (p0_ref[0] + p0_ref[1] + g0_ref[...]) * dis
    g1_ref[...] = jnp.maximum(s0, 0.0) * dis


def _fin_body(p1_ref, g1_ref, dacc_ref, w1t_ref, out_ref):
    dis = _dis_of(dacc_ref)
    s1 = (p1_ref[0] + p1_ref[1] + g1_ref[...]) * dis
    out_ref[...] = jnp.dot(s1, w1t_ref[...],
                           preferred_element_type=jnp.float32)


def _row_spec(d):
    return pl.BlockSpec((_BLK, d), lambda i: (i, 0))


def _pair_spec(d):
    return pl.BlockSpec((_NC, _BLK, d), lambda i: (0, i, 0))


def _full_spec(r, c):
    return pl.BlockSpec((r, c), lambda i: (0, 0))


def _mm0(X, W0T, dacc):
    return pl.pallas_call(
        _mm0_body,
        grid=(_N // _BLK,),
        in_specs=[_row_spec(_DF), _full_spec(_DF, _DF), _pair_spec(_DC)],
        out_specs=_row_spec(_DF),
        out_shape=jax.ShapeDtypeStruct((_N, _DF), jnp.float32),
    )(X, W0T, dacc)


def _mid(p0, g0, dacc):
    return pl.pallas_call(
        _mid_body,
        grid=(_N // _BLK,),
        in_specs=[_pair_spec(_DF), _row_spec(_DF), _pair_spec(_DC)],
        out_specs=_row_spec(_DF),
        out_shape=jax.ShapeDtypeStruct((_N, _DF), jnp.float32),
    )(p0, g0, dacc)


def _fin(p1, g1, dacc, W1T):
    return pl.pallas_call(
        _fin_body,
        grid=(_N // _BLK,),
        in_specs=[_pair_spec(_DF), _row_spec(_DF), _pair_spec(_DC),
                  _full_spec(_DF, _DC)],
        out_specs=_row_spec(_DC),
        out_shape=jax.ShapeDtypeStruct((_N, _DC), jnp.float32),
    )(p1, g1, dacc, W1T)


def kernel(X, edge_index, W0, W1):
    rows = edge_index[0].reshape(_NW, _NCHUNK, _CHUNK)
    cols = edge_index[1].reshape(_NW, _NCHUNK, _CHUNK)
    ones = jnp.ones((_CHUNK, _DC), jnp.float32)
    z16 = jnp.zeros((_NPAD, _DC), jnp.float32)
    z128 = jnp.zeros((_NPAD, _DF), jnp.float32)

    dacc = _hist(rows, ones, z16)
    g0 = _mm0(X, W0.T, dacc)
    p0 = _spmm128(rows, cols, g0, z128)
    g1 = _mid(p0, g0, dacc)
    p1 = _spmm128(rows, cols, g1, z128)
    return _fin(p1, g1, dacc, W1.T)
